# baseline (speedup 1.0000x reference)
import functools

import jax
import jax.numpy as jnp
import numpy as np
from jax import lax
from jax.experimental import pallas as pl
from jax.experimental.pallas import tpu as pltpu
from jax.experimental.pallas import tpu_sc as plsc

N_NODES = 10000
N_EDGES = 320000
IN_FEAT = 128
OUT_FEAT = 64
N_BASIS = 4
FF = OUT_FEAT * N_BASIS

NC = 2
NS = 16
NW = NC * NS
LANES = 16

EPW = N_EDGES // NW
CH = 40
NCHUNK = EPW // CH
GRP = CH // 4
NSLOT = 2

_k2 = np.arange(FF)
_PERM = 4 * (_k2 % OUT_FEAT) + (_k2 // OUT_FEAT)


def _node_tables(prop, wi, wj, bj):
    def mm(p_ref, wi_ref, wj_ref, b_ref, ti_ref, tj_ref):
        p = p_ref[...]
        ti_ref[...] = jnp.dot(p, wi_ref[...], preferred_element_type=jnp.float32)
        tj_ref[...] = (
            jnp.dot(p, wj_ref[...], preferred_element_type=jnp.float32)
            + b_ref[...]
        )

    rows = 2000
    grid = N_NODES // rows
    return pl.pallas_call(
        mm,
        grid=(grid,),
        in_specs=[
            pl.BlockSpec((rows, IN_FEAT), lambda i: (i, 0)),
            pl.BlockSpec((IN_FEAT, FF), lambda i: (0, 0)),
            pl.BlockSpec((IN_FEAT, FF), lambda i: (0, 0)),
            pl.BlockSpec((1, FF), lambda i: (0, 0)),
        ],
        out_specs=[
            pl.BlockSpec((rows, FF), lambda i: (i, 0)),
            pl.BlockSpec((rows, FF), lambda i: (i, 0)),
        ],
        out_shape=[
            jax.ShapeDtypeStruct((N_NODES, FF), jnp.float32),
            jax.ShapeDtypeStruct((N_NODES, FF), jnp.float32),
        ],
    )(prop, wi, wj, bj)


_SC_MESH = plsc.VectorSubcoreMesh(
    core_axis_name="c", subcore_axis_name="s", num_cores=NC, num_subcores=NS
)

_SCRATCH = (
    [pltpu.VMEM((2 * EPW,), jnp.int32)]
    + [pltpu.VMEM((EPW * N_BASIS,), jnp.float32)]
    + [pltpu.VMEM((2 * CH, FF), jnp.float32)] * NSLOT
    + [pltpu.VMEM((CH, OUT_FEAT), jnp.float32)] * NSLOT
    + [pltpu.SemaphoreType.DMA] * (2 * NSLOT)
)


@functools.partial(
    pl.kernel,
    out_type=jax.ShapeDtypeStruct((N_EDGES, OUT_FEAT), jnp.float32),
    mesh=_SC_MESH,
    scratch_types=_SCRATCH,
)
def _edge_kernel(tcat_hbm, ii2_hbm, bas_hbm, out_hbm, ii_v, bas_v, *rest):
    rows = rest[0:NSLOT]
    ov = rest[NSLOT:2 * NSLOT]
    sa = rest[2 * NSLOT:3 * NSLOT]
    so = rest[3 * NSLOT:4 * NSLOT]

    wid = lax.axis_index("s") * NC + lax.axis_index("c")
    base = wid * EPW

    pltpu.sync_copy(ii2_hbm.at[pl.ds(base * 2, 2 * EPW)], ii_v)
    pltpu.sync_copy(bas_hbm.at[pl.ds(base * N_BASIS, EPW * N_BASIS)], bas_v)

    def desc_a(chunk, slot):
        idx = ii_v.at[pl.ds(chunk * 2 * CH, 2 * CH)]
        return pltpu.make_async_copy(tcat_hbm.at[idx], rows[slot], sa[slot])

    def desc_o(chunk, slot):
        return pltpu.make_async_copy(
            ov[slot], out_hbm.at[pl.ds(base + chunk * CH, CH)], so[slot]
        )

    def compute(chunk, slot):
        rv, o = rows[slot], ov[slot]

        def splats(bgrp, eq):
            return [
                jnp.take_along_axis(
                    bgrp,
                    jnp.full((LANES,), eq * N_BASIS + q, jnp.int32),
                    axis=0,
                    mode="promise_in_bounds",
                )
                for q in range(N_BASIS)
            ]

        def load_edge(e):
            return (
                [rv[e, pl.ds(LANES * k, LANES)] for k in range(16)],
                [rv[CH + e, pl.ds(LANES * k, LANES)] for k in range(16)],
            )

        def compute_edge(e, ld, sp):
            sv = [a + bb for a, bb in zip(*ld)]
            accs = []
            for r in range(4):
                acc = None
                for q in range(N_BASIS):
                    t = sp[q] * sv[4 * q + r]
                    acc = t if acc is None else acc + t
                accs.append(acc)
            for r in range(4):
                o[e, pl.ds(LANES * r, LANES)] = accs[r]

        def grp_body(g, carry):
            bgrp = bas_v[pl.ds(chunk * (CH * N_BASIS) + g * LANES, LANES)]
            prev = None
            for eq in range(4):
                e = g * 4 + eq
                ld = load_edge(e)
                sp = splats(bgrp, eq)
                if prev is not None:
                    compute_edge(*prev)
                prev = (e, ld, sp)
            compute_edge(*prev)
            return carry

        lax.fori_loop(0, GRP, grp_body, 0)

    def body(chunk, b):
        @pl.when(chunk + NSLOT - 1 < NCHUNK)
        def _():
            desc_a(chunk + NSLOT - 1, (b + NSLOT - 1) % NSLOT).start()

        desc_a(chunk, b).wait()

        @pl.when(chunk >= NSLOT)
        def _():
            desc_o(chunk - NSLOT, b).wait()

        compute(chunk, b)
        desc_o(chunk, b).start()

    for c in range(NSLOT - 1):
        desc_a(c, c).start()

    def ring_body(cg, carry):
        for b in range(NSLOT):
            body(cg * NSLOT + b, b)
        return carry

    lax.fori_loop(0, NCHUNK // NSLOT, ring_body, 0)
    for c in range(NCHUNK - NCHUNK % NSLOT, NCHUNK):
        body(jnp.int32(c), c % NSLOT)
    for c in range(NCHUNK - NSLOT, NCHUNK):
        desc_o(c, c % NSLOT).wait()


def kernel(prop, idx_i, idx_j, basis, W, b):
    W = W.astype(jnp.float32)
    wp = W[:, _PERM]
    bp = b.astype(jnp.float32)[_PERM].reshape(1, FF)
    ti, tj = _node_tables(
        prop.astype(jnp.float32), wp[:IN_FEAT], wp[IN_FEAT:], bp
    )
    tcat = jnp.concatenate([ti, tj], axis=0)
    ii2 = jnp.concatenate(
        [
            idx_i.astype(jnp.int32).reshape(-1, CH),
            idx_j.astype(jnp.int32).reshape(-1, CH) + N_NODES,
        ],
        axis=1,
    ).reshape(-1)
    out = _edge_kernel(
        tcat,
        ii2,
        basis.astype(jnp.float32).reshape(-1),
    )
    return out


# speedup vs baseline: 1.0791x; 1.0791x over previous
import functools

import jax
import jax.numpy as jnp
import numpy as np
from jax import lax
from jax.experimental import pallas as pl
from jax.experimental.pallas import tpu as pltpu
from jax.experimental.pallas import tpu_sc as plsc

N_NODES = 10000
N_EDGES = 320000
IN_FEAT = 128
OUT_FEAT = 64
N_BASIS = 4
FF = OUT_FEAT * N_BASIS

NC = 2
NS = 16
NW = NC * NS
LANES = 16

EPW = N_EDGES // NW
CH = 40
NCHUNK = EPW // CH
GRP = CH // 4
NSLOT = 2

_k2 = np.arange(FF)
_PERM = 4 * (_k2 % OUT_FEAT) + (_k2 // OUT_FEAT)


def _node_tables(prop, wi, wj, bj):
    def mm(p_ref, wi_ref, wj_ref, b_ref, ti_ref, tj_ref):
        p = p_ref[...]
        ti_ref[...] = jnp.dot(p, wi_ref[...], preferred_element_type=jnp.float32)
        tj_ref[...] = (
            jnp.dot(p, wj_ref[...], preferred_element_type=jnp.float32)
            + b_ref[...]
        )

    rows = 2000
    grid = N_NODES // rows
    return pl.pallas_call(
        mm,
        grid=(grid,),
        in_specs=[
            pl.BlockSpec((rows, IN_FEAT), lambda i: (i, 0)),
            pl.BlockSpec((IN_FEAT, FF), lambda i: (0, 0)),
            pl.BlockSpec((IN_FEAT, FF), lambda i: (0, 0)),
            pl.BlockSpec((1, FF), lambda i: (0, 0)),
        ],
        out_specs=[
            pl.BlockSpec((rows, FF), lambda i: (i, 0)),
            pl.BlockSpec((rows, FF), lambda i: (i, 0)),
        ],
        out_shape=[
            jax.ShapeDtypeStruct((N_NODES, FF), jnp.float32),
            jax.ShapeDtypeStruct((N_NODES, FF), jnp.float32),
        ],
    )(prop, wi, wj, bj)


_SC_MESH = plsc.VectorSubcoreMesh(
    core_axis_name="c", subcore_axis_name="s", num_cores=NC, num_subcores=NS
)

_SCRATCH = (
    [pltpu.VMEM((EPW,), jnp.int32)] * 2
    + [pltpu.VMEM((EPW * N_BASIS,), jnp.float32)]
    + [pltpu.VMEM((CH, FF), jnp.float32)] * (2 * NSLOT)
    + [pltpu.VMEM((CH, OUT_FEAT), jnp.float32)] * NSLOT
    + [pltpu.SemaphoreType.DMA] * (3 * NSLOT)
)


@functools.partial(
    pl.kernel,
    out_type=jax.ShapeDtypeStruct((N_EDGES, OUT_FEAT), jnp.float32),
    mesh=_SC_MESH,
    scratch_types=_SCRATCH,
)
def _edge_kernel(ti_hbm, tj_hbm, ii_hbm, jj_hbm, bas_hbm, out_hbm,
                 ii_v, jj_v, bas_v, *rest):
    ri = rest[0:NSLOT]
    rj = rest[NSLOT:2 * NSLOT]
    ov = rest[2 * NSLOT:3 * NSLOT]
    sa = rest[3 * NSLOT:4 * NSLOT]
    sb = rest[4 * NSLOT:5 * NSLOT]
    so = rest[5 * NSLOT:6 * NSLOT]

    wid = lax.axis_index("s") * NC + lax.axis_index("c")
    base = wid * EPW

    pltpu.sync_copy(ii_hbm.at[pl.ds(base, EPW)], ii_v)
    pltpu.sync_copy(jj_hbm.at[pl.ds(base, EPW)], jj_v)
    pltpu.sync_copy(bas_hbm.at[pl.ds(base * N_BASIS, EPW * N_BASIS)], bas_v)

    def desc_a(chunk, slot):
        idx_i = ii_v.at[pl.ds(chunk * CH, CH)]
        return pltpu.make_async_copy(ti_hbm.at[idx_i], ri[slot], sa[slot])

    def desc_b(chunk, slot):
        idx_j = jj_v.at[pl.ds(chunk * CH, CH)]
        return pltpu.make_async_copy(tj_hbm.at[idx_j], rj[slot], sb[slot])

    def desc_o(chunk, slot):
        return pltpu.make_async_copy(
            ov[slot], out_hbm.at[pl.ds(base + chunk * CH, CH)], so[slot]
        )

    def compute(chunk, slot):
        riv, rjv, o = ri[slot], rj[slot], ov[slot]

        def splats(bgrp, eq):
            return [
                jnp.take_along_axis(
                    bgrp,
                    jnp.full((LANES,), eq * N_BASIS + q, jnp.int32),
                    axis=0,
                    mode="promise_in_bounds",
                )
                for q in range(N_BASIS)
            ]

        def load_edge(e):
            return (
                [riv[e, pl.ds(LANES * k, LANES)] for k in range(16)],
                [rjv[e, pl.ds(LANES * k, LANES)] for k in range(16)],
            )

        def compute_edge(e, ld, sp):
            sv = [a + bb for a, bb in zip(*ld)]
            accs = []
            for r in range(4):
                acc = None
                for q in range(N_BASIS):
                    t = sp[q] * sv[4 * q + r]
                    acc = t if acc is None else acc + t
                accs.append(acc)
            for r in range(4):
                o[e, pl.ds(LANES * r, LANES)] = accs[r]

        def grp_body(g, carry):
            bgrp = bas_v[pl.ds(chunk * (CH * N_BASIS) + g * LANES, LANES)]
            prev = None
            for eq in range(4):
                e = g * 4 + eq
                ld = load_edge(e)
                sp = splats(bgrp, eq)
                if prev is not None:
                    compute_edge(*prev)
                prev = (e, ld, sp)
            compute_edge(*prev)
            return carry

        lax.fori_loop(0, GRP, grp_body, 0)

    def body(chunk, b):
        @pl.when(chunk + NSLOT - 1 < NCHUNK)
        def _():
            desc_a(chunk + NSLOT - 1, (b + NSLOT - 1) % NSLOT).start()
            desc_b(chunk + NSLOT - 1, (b + NSLOT - 1) % NSLOT).start()

        desc_a(chunk, b).wait()
        desc_b(chunk, b).wait()

        @pl.when(chunk >= NSLOT)
        def _():
            desc_o(chunk - NSLOT, b).wait()

        compute(chunk, b)
        desc_o(chunk, b).start()

    for c in range(NSLOT - 1):
        desc_a(c, c).start()
        desc_b(c, c).start()

    def ring_body(cg, carry):
        for b in range(NSLOT):
            body(cg * NSLOT + b, b)
        return carry

    lax.fori_loop(0, NCHUNK // NSLOT, ring_body, 0)
    for c in range(NCHUNK - NCHUNK % NSLOT, NCHUNK):
        body(jnp.int32(c), c % NSLOT)
    for c in range(NCHUNK - NSLOT, NCHUNK):
        desc_o(c, c % NSLOT).wait()


def kernel(prop, idx_i, idx_j, basis, W, b):
    W = W.astype(jnp.float32)
    wp = W[:, _PERM]
    bp = b.astype(jnp.float32)[_PERM].reshape(1, FF)
    ti, tj = _node_tables(
        prop.astype(jnp.float32), wp[:IN_FEAT], wp[IN_FEAT:], bp
    )
    out = _edge_kernel(
        ti,
        tj,
        idx_i.astype(jnp.int32),
        idx_j.astype(jnp.int32),
        basis.astype(jnp.float32).reshape(-1),
    )
    return out


# speedup vs baseline: 1.0823x; 1.0029x over previous
import functools

import jax
import jax.numpy as jnp
import numpy as np
from jax import lax
from jax.experimental import pallas as pl
from jax.experimental.pallas import tpu as pltpu
from jax.experimental.pallas import tpu_sc as plsc

N_NODES = 10000
N_EDGES = 320000
IN_FEAT = 128
OUT_FEAT = 64
N_BASIS = 4
FF = OUT_FEAT * N_BASIS

NC = 2
NS = 16
NW = NC * NS
LANES = 16

EPW = N_EDGES // NW
CH = 40
NCHUNK = EPW // CH
GRP = CH // 4
NSLOT = 2

_k2 = np.arange(FF)
_PERM = 4 * (_k2 % OUT_FEAT) + (_k2 // OUT_FEAT)


def _node_tables(prop, wi, wj, bj):
    def mm(p_ref, wi_ref, wj_ref, b_ref, ti_ref, tj_ref):
        p = p_ref[...]
        ti_ref[...] = jnp.dot(p, wi_ref[...], preferred_element_type=jnp.float32)
        tj_ref[...] = (
            jnp.dot(p, wj_ref[...], preferred_element_type=jnp.float32)
            + b_ref[...]
        )

    rows = 2000
    grid = N_NODES // rows
    return pl.pallas_call(
        mm,
        grid=(grid,),
        in_specs=[
            pl.BlockSpec((rows, IN_FEAT), lambda i: (i, 0)),
            pl.BlockSpec((IN_FEAT, FF), lambda i: (0, 0)),
            pl.BlockSpec((IN_FEAT, FF), lambda i: (0, 0)),
            pl.BlockSpec((1, FF), lambda i: (0, 0)),
        ],
        out_specs=[
            pl.BlockSpec((rows, FF), lambda i: (i, 0)),
            pl.BlockSpec((rows, FF), lambda i: (i, 0)),
        ],
        out_shape=[
            jax.ShapeDtypeStruct((N_NODES, FF), jnp.float32),
            jax.ShapeDtypeStruct((N_NODES, FF), jnp.float32),
        ],
    )(prop, wi, wj, bj)


_SC_MESH = plsc.VectorSubcoreMesh(
    core_axis_name="c", subcore_axis_name="s", num_cores=NC, num_subcores=NS
)

NIDX = 4

_SCRATCH = (
    [pltpu.VMEM((CH,), jnp.int32)] * (2 * NIDX)
    + [pltpu.VMEM((CH * N_BASIS,), jnp.float32)] * NIDX
    + [pltpu.VMEM((CH, FF), jnp.float32)] * (2 * NSLOT)
    + [pltpu.VMEM((CH, OUT_FEAT), jnp.float32)] * NSLOT
    + [pltpu.SemaphoreType.DMA] * (NIDX + 3 * NSLOT)
)


@functools.partial(
    pl.kernel,
    out_type=jax.ShapeDtypeStruct((N_EDGES, OUT_FEAT), jnp.float32),
    mesh=_SC_MESH,
    scratch_types=_SCRATCH,
)
def _edge_kernel(ti_hbm, tj_hbm, ii_hbm, jj_hbm, bas_hbm, out_hbm, *rest):
    ii_r = rest[0:NIDX]
    jj_r = rest[NIDX:2 * NIDX]
    ba_r = rest[2 * NIDX:3 * NIDX]
    rest = rest[3 * NIDX:]
    ri = rest[0:NSLOT]
    rj = rest[NSLOT:2 * NSLOT]
    ov = rest[2 * NSLOT:3 * NSLOT]
    si = rest[3 * NSLOT:3 * NSLOT + NIDX]
    rest2 = rest[3 * NSLOT + NIDX:]
    sa = rest2[0:NSLOT]
    sb = rest2[NSLOT:2 * NSLOT]
    so = rest2[2 * NSLOT:3 * NSLOT]

    wid = lax.axis_index("s") * NC + lax.axis_index("c")
    base = wid * EPW

    def idx_descs(chunk, islot):
        off = base + chunk * CH
        return (
            pltpu.make_async_copy(
                ii_hbm.at[pl.ds(off, CH)], ii_r[islot], si[islot]
            ),
            pltpu.make_async_copy(
                jj_hbm.at[pl.ds(off, CH)], jj_r[islot], si[islot]
            ),
            pltpu.make_async_copy(
                bas_hbm.at[pl.ds(off * N_BASIS, CH * N_BASIS)],
                ba_r[islot],
                si[islot],
            ),
        )

    def desc_a(chunk, slot, islot):
        return pltpu.make_async_copy(ti_hbm.at[ii_r[islot]], ri[slot], sa[slot])

    def desc_b(chunk, slot, islot):
        return pltpu.make_async_copy(tj_hbm.at[jj_r[islot]], rj[slot], sb[slot])

    def desc_o(chunk, slot):
        return pltpu.make_async_copy(
            ov[slot], out_hbm.at[pl.ds(base + chunk * CH, CH)], so[slot]
        )

    def compute(chunk, slot, islot):
        riv, rjv, o = ri[slot], rj[slot], ov[slot]
        bas_v = ba_r[islot]

        def splats(bgrp, eq):
            return [
                jnp.take_along_axis(
                    bgrp,
                    jnp.full((LANES,), eq * N_BASIS + q, jnp.int32),
                    axis=0,
                    mode="promise_in_bounds",
                )
                for q in range(N_BASIS)
            ]

        def load_edge(e):
            return (
                [riv[e, pl.ds(LANES * k, LANES)] for k in range(16)],
                [rjv[e, pl.ds(LANES * k, LANES)] for k in range(16)],
            )

        def compute_edge(e, ld, sp):
            sv = [a + bb for a, bb in zip(*ld)]
            accs = []
            for r in range(4):
                acc = None
                for q in range(N_BASIS):
                    t = sp[q] * sv[4 * q + r]
                    acc = t if acc is None else acc + t
                accs.append(acc)
            for r in range(4):
                o[e, pl.ds(LANES * r, LANES)] = accs[r]

        def grp_body(g, carry):
            bgrp = bas_v[pl.ds(g * LANES, LANES)]
            prev = None
            for eq in range(4):
                e = g * 4 + eq
                ld = load_edge(e)
                sp = splats(bgrp, eq)
                if prev is not None:
                    compute_edge(*prev)
                prev = (e, ld, sp)
            compute_edge(*prev)
            return carry

        lax.fori_loop(0, GRP, grp_body, 0)

    def body(chunk, b, ib):
        @pl.when(chunk + 2 < NCHUNK)
        def _():
            for d in idx_descs(chunk + 2, (ib + 2) % NIDX):
                d.start()

        @pl.when(chunk + 1 < NCHUNK)
        def _():
            for d in idx_descs(chunk + 1, (ib + 1) % NIDX):
                d.wait()
            desc_a(chunk + 1, 1 - b, (ib + 1) % NIDX).start()
            desc_b(chunk + 1, 1 - b, (ib + 1) % NIDX).start()

        desc_a(chunk, b, ib).wait()
        desc_b(chunk, b, ib).wait()

        @pl.when(chunk >= NSLOT)
        def _():
            desc_o(chunk - NSLOT, b).wait()

        compute(chunk, b, ib)
        desc_o(chunk, b).start()

    for d in idx_descs(0, 0):
        d.start()
    for d in idx_descs(1, 1):
        d.start()
    for d in idx_descs(0, 0):
        d.wait()
    desc_a(0, 0, 0).start()
    desc_b(0, 0, 0).start()

    def ring_body(cg, carry):
        for k in range(NIDX):
            body(cg * NIDX + k, k % NSLOT, k)
        return carry

    lax.fori_loop(0, NCHUNK // NIDX, ring_body, 0)
    for c in range(NCHUNK - NCHUNK % NIDX, NCHUNK):
        body(jnp.int32(c), c % NSLOT, c % NIDX)
    for c in range(NCHUNK - NSLOT, NCHUNK):
        desc_o(c, c % NSLOT).wait()


def kernel(prop, idx_i, idx_j, basis, W, b):
    W = W.astype(jnp.float32)
    wp = W[:, _PERM]
    bp = b.astype(jnp.float32)[_PERM].reshape(1, FF)
    ti, tj = _node_tables(
        prop.astype(jnp.float32), wp[:IN_FEAT], wp[IN_FEAT:], bp
    )
    out = _edge_kernel(
        ti,
        tj,
        idx_i.astype(jnp.int32),
        idx_j.astype(jnp.int32),
        basis.astype(jnp.float32).reshape(-1),
    )
    return out
